# baseline (speedup 1.0000x reference)
import jax
import jax.numpy as jnp
from jax import lax
from jax.experimental import pallas as pl
from jax.experimental.pallas import tpu as pltpu

_N = 100000
_D = 128
_BLOCK = 2000


def _body(keys_ref, q_ref, out_ref, min_ref, idx_ref):
    pid = pl.program_id(0)

    @pl.when(pid == 0)
    def _init():
        min_ref[0] = jnp.float32(jnp.inf)
        idx_ref[0] = jnp.int32(0)

    x = jnp.abs(keys_ref[...] - q_ref[...])
    s = jnp.sum(x, axis=1)
    m = jnp.min(s)
    iota = lax.broadcasted_iota(jnp.int32, (_BLOCK,), 0)
    local_idx = jnp.min(jnp.where(s == m, iota, jnp.int32(_N)))

    @pl.when(m < min_ref[0])
    def _update():
        min_ref[0] = m
        idx_ref[0] = pid * _BLOCK + local_idx

    @pl.when(pid == pl.num_programs(0) - 1)
    def _emit():
        out_ref[0] = idx_ref[0]


def kernel(keys, query):
    grid = _N // _BLOCK
    out = pl.pallas_call(
        _body,
        grid=(grid,),
        in_specs=[
            pl.BlockSpec((_BLOCK, _D), lambda i: (i, 0)),
            pl.BlockSpec((1, _D), lambda i: (0, 0)),
        ],
        out_specs=pl.BlockSpec(memory_space=pltpu.SMEM),
        out_shape=jax.ShapeDtypeStruct((1,), jnp.int32),
        scratch_shapes=[
            pltpu.SMEM((1,), jnp.float32),
            pltpu.SMEM((1,), jnp.int32),
        ],
    )(keys, query)
    return out[0]
